# baseline (speedup 1.0000x reference)
import jax
import jax.numpy as jnp
from jax import lax
from jax.experimental import pallas as pl

_K = 512
_BINS = _K + 1


def _body(x_ref, c_ref, o_ref):
    cm = c_ref[...]
    cn = jnp.sum(cm * cm, axis=1, keepdims=True)
    for b in range(4):
        xb = x_ref[pl.ds(b * 96, 96), :]
        prod = lax.dot_general(
            cm, xb, (((1,), (0,)), ((), ())),
            precision=lax.Precision.HIGHEST,
            preferred_element_type=jnp.float32,
        )
        s = cn - 2.0 * prod
        m = jnp.min(s, axis=0, keepdims=True)
        ki = lax.broadcasted_iota(jnp.int32, s.shape, 0)
        code = jnp.min(jnp.where(s == m, ki, _K), axis=0, keepdims=True).T
        bins = lax.broadcasted_iota(jnp.int32, (256, _K), 1)
        oh = (bins == code).astype(jnp.float32)
        hist = oh.reshape(4, 4, 4, 4, _K).sum(axis=(1, 3)).reshape(16, _K)
        full = jnp.pad(hist * (1.0 / 16.0), ((0, 0), (1, 0)))
        o_ref[pl.ds(b * 16, 16), :] = full


def kernel(x, cluster_centers):
    B, C, H, W = x.shape
    xr = x.reshape(B * C, H * W)
    cm = cluster_centers.reshape(_K, C)
    return pl.pallas_call(
        _body,
        out_shape=jax.ShapeDtypeStruct((B * 16, _BINS), jnp.float32),
    )(xr, cm).reshape(B, 16, _BINS)
